# baseline (speedup 1.0000x reference)
import functools

import jax
import jax.numpy as jnp
from jax import lax
from jax.experimental import pallas as pl
from jax.experimental.pallas import tpu as pltpu
from jax.experimental.pallas import tpu_sc as plsc

N_NODES = 10000
N_EDGES = 320000
D = 128
H = 128

EB = 128
NB = N_EDGES // EB


def _edge_body(ef_ref, g_ref, w1_ref, b1_ref, w2_ref, b2_ref,
               gm_ref, bt_ref, out_ref):
    x = ef_ref[...]
    h = jnp.dot(x, w1_ref[...], preferred_element_type=jnp.float32)
    h = h + g_ref[...] + b1_ref[...]
    h = h * jax.nn.sigmoid(h)
    y = jnp.dot(h, w2_ref[...], preferred_element_type=jnp.float32) + b2_ref[...]
    mu = jnp.mean(y, axis=-1, keepdims=True)
    var = jnp.mean((y - mu) ** 2, axis=-1, keepdims=True)
    y = (y - mu) * lax.rsqrt(var + 1e-5)
    out_ref[...] = y * gm_ref[...] + bt_ref[...] + x


def _edge_mlp(ef, g, w1e, b1, w2, b2, gamma, beta, blk=8000):
    grid = (N_EDGES // blk,)
    row = lambda i: (i, 0)
    fix = lambda i: (0, 0)
    return pl.pallas_call(
        _edge_body,
        grid=grid,
        in_specs=[
            pl.BlockSpec((blk, D), row),
            pl.BlockSpec((blk, H), row),
            pl.BlockSpec((D, H), fix),
            pl.BlockSpec((1, H), fix),
            pl.BlockSpec((H, D), fix),
            pl.BlockSpec((1, D), fix),
            pl.BlockSpec((1, D), fix),
            pl.BlockSpec((1, D), fix),
        ],
        out_specs=pl.BlockSpec((blk, D), row),
        out_shape=jax.ShapeDtypeStruct((N_EDGES, D), jnp.float32),
        compiler_params=pltpu.CompilerParams(
            dimension_semantics=("arbitrary",)),
    )(ef, g, w1e, b1, w2, b2, gamma, beta)


def _node_body(nf_ref, agg_ref, w1n_ref, w1a_ref, b1_ref, w2_ref, b2_ref,
               gm_ref, bt_ref, w1s_ref, w1d_ref,
               nf_out, ps_out, pd_out):
    x = nf_ref[...]
    a = agg_ref[0] + agg_ref[1]
    h = (jnp.dot(x, w1n_ref[...], preferred_element_type=jnp.float32)
         + jnp.dot(a, w1a_ref[...], preferred_element_type=jnp.float32)
         + b1_ref[...])
    h = h * jax.nn.sigmoid(h)
    y = jnp.dot(h, w2_ref[...], preferred_element_type=jnp.float32) + b2_ref[...]
    mu = jnp.mean(y, axis=-1, keepdims=True)
    var = jnp.mean((y - mu) ** 2, axis=-1, keepdims=True)
    y = (y - mu) * lax.rsqrt(var + 1e-5)
    y = y * gm_ref[...] + bt_ref[...] + x
    nf_out[...] = y
    ps_out[...] = jnp.dot(y, w1s_ref[...], preferred_element_type=jnp.float32)
    pd_out[...] = jnp.dot(y, w1d_ref[...], preferred_element_type=jnp.float32)


def _node_mlp(nf, agg2, w1n, w1a, b1, w2, b2, gamma, beta, w1s_nxt, w1d_nxt):
    out_shape = [
        jax.ShapeDtypeStruct((N_NODES, D), jnp.float32),
        jax.ShapeDtypeStruct((N_NODES, H), jnp.float32),
        jax.ShapeDtypeStruct((N_NODES, H), jnp.float32),
    ]
    return pl.pallas_call(_node_body, out_shape=out_shape)(
        nf, agg2, w1n, w1a, b1, w2, b2, gamma, beta, w1s_nxt, w1d_nxt)


def _proj_body(nf_ref, w1s_ref, w1d_ref, ps_out, pd_out):
    x = nf_ref[...]
    ps_out[...] = jnp.dot(x, w1s_ref[...], preferred_element_type=jnp.float32)
    pd_out[...] = jnp.dot(x, w1d_ref[...], preferred_element_type=jnp.float32)


def _proj(nf, w1s, w1d):
    out_shape = [
        jax.ShapeDtypeStruct((N_NODES, H), jnp.float32),
        jax.ShapeDtypeStruct((N_NODES, H), jnp.float32),
    ]
    return pl.pallas_call(_proj_body, out_shape=out_shape)(nf, w1s, w1d)


def _make_gather():
    info = plsc.get_sparse_core_info()
    nc, ns = info.num_cores, info.num_subcores
    nw = nc * ns
    mesh = plsc.VectorSubcoreMesh(core_axis_name="c", subcore_axis_name="s")
    GB = 200
    E_PER = N_EDGES // nw
    NIT = E_PER // GB
    SUB = ((0, 128), (128, 72))
    CHUNKS = GB * H // 16

    @functools.partial(
        pl.kernel,
        mesh=mesh,
        out_type=jax.ShapeDtypeStruct((N_EDGES, H), jnp.float32),
        scratch_types=[
            pltpu.VMEM((E_PER,), jnp.int32),
            pltpu.VMEM((E_PER,), jnp.int32),
            pltpu.VMEM((GB, H), jnp.float32),
            pltpu.VMEM((GB, H), jnp.float32),
            pltpu.VMEM((GB, H), jnp.float32),
            pltpu.VMEM((GB, H), jnp.float32),
            pltpu.SemaphoreType.DMA,
            pltpu.SemaphoreType.DMA,
            pltpu.SemaphoreType.DMA,
            pltpu.SemaphoreType.DMA,
        ],
    )
    def gather(ps_hbm, pd_hbm, src_hbm, dst_hbm, g_hbm,
               isrc, idst, a0, d0, a1, d1, g0, g1, w0, w1):
        cid = lax.axis_index("c")
        sid = lax.axis_index("s")
        wid = sid * nc + cid
        base_e = wid * E_PER
        abufs = (a0, a1)
        dbufs = (d0, d1)
        gsems = (g0, g1)
        wsems = (w0, w1)

        pltpu.sync_copy(src_hbm.at[pl.ds(base_e, E_PER)], isrc)
        pltpu.sync_copy(dst_hbm.at[pl.ds(base_e, E_PER)], idst)

        def fire(it, s):
            off = it * GB
            for (o, n) in SUB:
                pltpu.async_copy(ps_hbm.at[isrc.at[pl.ds(off + o, n)]],
                                 abufs[s].at[pl.ds(o, n)], gsems[s])
                pltpu.async_copy(pd_hbm.at[idst.at[pl.ds(off + o, n)]],
                                 dbufs[s].at[pl.ds(o, n)], gsems[s])

        def drain(it, s):
            for (o, n) in SUB:
                pltpu.make_async_copy(ps_hbm.at[pl.ds(0, n)],
                                      abufs[s].at[pl.ds(o, n)],
                                      gsems[s]).wait()
                pltpu.make_async_copy(pd_hbm.at[pl.ds(0, n)],
                                      dbufs[s].at[pl.ds(o, n)],
                                      gsems[s]).wait()
            a, dd = abufs[s], dbufs[s]

            def add_body(t, _):
                r = t >> 3
                c = (t & 7) * 16
                a[r, pl.ds(c, 16)] = a[r, pl.ds(c, 16)] + dd[r, pl.ds(c, 16)]
                return 0

            lax.fori_loop(0, CHUNKS, add_body, 0)
            pltpu.async_copy(a, g_hbm.at[pl.ds(base_e + it * GB, GB)],
                             wsems[s])

        def wait_w(s):
            pltpu.make_async_copy(g_hbm.at[pl.ds(base_e, GB)],
                                  abufs[s], wsems[s]).wait()

        fire(0, 0)

        def body(i, _):
            @pl.when(i > 0)
            def _():
                wait_w(1)

            fire(2 * i + 1, 1)
            drain(2 * i, 0)

            @pl.when(i < NIT // 2 - 1)
            def _():
                wait_w(0)
                fire(2 * i + 2, 0)

            drain(2 * i + 1, 1)
            return 0

        lax.fori_loop(0, NIT // 2, body, 0)
        wait_w(0)
        wait_w(1)

    return gather


def _make_scatter():
    info = plsc.get_sparse_core_info()
    nc, ns = info.num_cores, info.num_subcores
    nw = nc * ns
    rps = 624
    tail = N_NODES - rps * ns
    mesh = plsc.VectorSubcoreMesh(core_axis_name="c", subcore_axis_name="s")

    blk_per_w = NB // nw
    n_extra = NB - blk_per_w * nw

    @functools.partial(
        pl.kernel,
        mesh=mesh,
        out_type=jax.ShapeDtypeStruct((2, N_NODES, D), jnp.float32),
        scratch_types=[
            pltpu.VMEM((EB,), jnp.int32),
            pltpu.VMEM((EB,), jnp.int32),
            pltpu.VMEM((EB, D), jnp.float32),
            pltpu.VMEM((EB, D), jnp.float32),
            pltpu.VMEM_SHARED((N_NODES, D), jnp.float32),
            pltpu.SemaphoreType.DMA,
            pltpu.SemaphoreType.DMA,
        ],
    )
    def scatter(e_hbm, dst_hbm, zeros_hbm, out_hbm,
                di0, di1, rb0, rb1, acc, r0sem, r1sem):
        cid = lax.axis_index("c")
        sid = lax.axis_index("s")
        wid = sid * nc + cid
        r0 = sid * rps
        pltpu.sync_copy(zeros_hbm.at[pl.ds(r0, rps)], acc.at[pl.ds(r0, rps)])

        @pl.when(sid == 0)
        def _():
            pltpu.sync_copy(zeros_hbm.at[pl.ds(rps * ns, tail)],
                            acc.at[pl.ds(rps * ns, tail)])

        plsc.subcore_barrier()

        t0 = wid * blk_per_w

        def fire(t, di, rb, rsem):
            pltpu.async_copy(dst_hbm.at[pl.ds(t * EB, EB)], di, rsem)
            pltpu.async_copy(e_hbm.at[pl.ds(t * EB, EB)], rb, rsem)

        def scat(di, rb, rsem):
            pltpu.make_async_copy(dst_hbm.at[pl.ds(0, EB)], di, rsem).wait()
            pltpu.make_async_copy(e_hbm.at[pl.ds(0, EB)], rb, rsem).wait()
            pltpu.sync_copy(rb, acc.at[di], add=True)

        fire(t0, di0, rb0, r0sem)

        def body(i, _):
            fire(t0 + 2 * i + 1, di1, rb1, r1sem)
            scat(di0, rb0, r0sem)

            @pl.when(i < blk_per_w // 2 - 1)
            def _():
                fire(t0 + 2 * i + 2, di0, rb0, r0sem)

            scat(di1, rb1, r1sem)
            return 0

        lax.fori_loop(0, blk_per_w // 2, body, 0)

        @pl.when(wid < n_extra)
        def _():
            fire(nw * blk_per_w + wid, di0, rb0, r0sem)
            scat(di0, rb0, r0sem)

        plsc.subcore_barrier()
        pltpu.sync_copy(acc.at[pl.ds(r0, rps)],
                        out_hbm.at[cid, pl.ds(r0, rps)])

        @pl.when(sid == 0)
        def _():
            pltpu.sync_copy(acc.at[pl.ds(rps * ns, tail)],
                            out_hbm.at[cid, pl.ds(rps * ns, tail)])

    return scatter


def kernel(efeat, nfeat, edge_index, params):
    src = edge_index[0].astype(jnp.int32)
    dst = edge_index[1].astype(jnp.int32)

    gather = _make_gather()
    scatter = _make_scatter()
    zeros = jnp.zeros((N_NODES, D), jnp.float32)

    def prep(p):
        e, n = p['edge'], p['node']
        return dict(
            w1e=e['w1'][:D], w1s=e['w1'][D:2 * D], w1d=e['w1'][2 * D:],
            eb1=e['b1'].reshape(1, H), ew2=e['w2'],
            eb2=e['b2'].reshape(1, D), eg=e['gamma'].reshape(1, D),
            ebt=e['beta'].reshape(1, D),
            w1n=n['w1'][:D], w1a=n['w1'][D:],
            nb1=n['b1'].reshape(1, H), nw2=n['w2'],
            nb2=n['b2'].reshape(1, D), ng=n['gamma'].reshape(1, D),
            nbt=n['beta'].reshape(1, D),
        )

    ps_list = [prep(p) for p in params]
    nlayers = len(ps_list)

    ps, pd = _proj(nfeat, ps_list[0]['w1s'], ps_list[0]['w1d'])
    for l, q in enumerate(ps_list):
        g = gather(ps, pd, src, dst)
        efeat = _edge_mlp(efeat, g, q['w1e'], q['eb1'], q['ew2'],
                          q['eb2'], q['eg'], q['ebt'])
        agg2 = scatter(efeat, dst, zeros)
        nxt = ps_list[(l + 1) % nlayers]
        nfeat, ps, pd = _node_mlp(nfeat, agg2, q['w1n'], q['w1a'], q['nb1'],
                                  q['nw2'], q['nb2'], q['ng'], q['nbt'],
                                  nxt['w1s'], nxt['w1d'])
    return (efeat, nfeat)


# speedup vs baseline: 1.5040x; 1.5040x over previous
import functools

import jax
import jax.numpy as jnp
from jax import lax
from jax.experimental import pallas as pl
from jax.experimental.pallas import tpu as pltpu
from jax.experimental.pallas import tpu_sc as plsc

N_NODES = 10000
N_EDGES = 320000
D = 128
H = 128

EB = 128
NB = N_EDGES // EB


def _edge_body(ef_ref, g_ref, w1_ref, b1_ref, w2_ref, b2_ref,
               gm_ref, bt_ref, out_ref):
    x = ef_ref[...]
    h = jnp.dot(x, w1_ref[...], preferred_element_type=jnp.float32)
    h = h + g_ref[...] + b1_ref[...]
    h = h * jax.nn.sigmoid(h)
    y = jnp.dot(h, w2_ref[...], preferred_element_type=jnp.float32) + b2_ref[...]
    mu = jnp.mean(y, axis=-1, keepdims=True)
    var = jnp.mean((y - mu) ** 2, axis=-1, keepdims=True)
    y = (y - mu) * lax.rsqrt(var + 1e-5)
    out_ref[...] = y * gm_ref[...] + bt_ref[...] + x


def _edge_mlp(ef, g, w1e, b1, w2, b2, gamma, beta, blk=8000):
    grid = (N_EDGES // blk,)
    row = lambda i: (i, 0)
    fix = lambda i: (0, 0)
    return pl.pallas_call(
        _edge_body,
        grid=grid,
        in_specs=[
            pl.BlockSpec((blk, D), row),
            pl.BlockSpec((blk, H), row),
            pl.BlockSpec((D, H), fix),
            pl.BlockSpec((1, H), fix),
            pl.BlockSpec((H, D), fix),
            pl.BlockSpec((1, D), fix),
            pl.BlockSpec((1, D), fix),
            pl.BlockSpec((1, D), fix),
        ],
        out_specs=pl.BlockSpec((blk, D), row),
        out_shape=jax.ShapeDtypeStruct((N_EDGES, D), jnp.float32),
        compiler_params=pltpu.CompilerParams(
            dimension_semantics=("arbitrary",)),
    )(ef, g, w1e, b1, w2, b2, gamma, beta)


def _node_body(nf_ref, agg_ref, w1n_ref, w1a_ref, b1_ref, w2_ref, b2_ref,
               gm_ref, bt_ref, w1s_ref, w1d_ref,
               nf_out, ps_out, pd_out):
    x = nf_ref[...]
    a = agg_ref[0] + agg_ref[1]
    h = (jnp.dot(x, w1n_ref[...], preferred_element_type=jnp.float32)
         + jnp.dot(a, w1a_ref[...], preferred_element_type=jnp.float32)
         + b1_ref[...])
    h = h * jax.nn.sigmoid(h)
    y = jnp.dot(h, w2_ref[...], preferred_element_type=jnp.float32) + b2_ref[...]
    mu = jnp.mean(y, axis=-1, keepdims=True)
    var = jnp.mean((y - mu) ** 2, axis=-1, keepdims=True)
    y = (y - mu) * lax.rsqrt(var + 1e-5)
    y = y * gm_ref[...] + bt_ref[...] + x
    nf_out[...] = y
    ps_out[...] = jnp.dot(y, w1s_ref[...], preferred_element_type=jnp.float32)
    pd_out[...] = jnp.dot(y, w1d_ref[...], preferred_element_type=jnp.float32)


def _node_mlp(nf, agg2, w1n, w1a, b1, w2, b2, gamma, beta, w1s_nxt, w1d_nxt):
    out_shape = [
        jax.ShapeDtypeStruct((N_NODES, D), jnp.float32),
        jax.ShapeDtypeStruct((N_NODES, H), jnp.float32),
        jax.ShapeDtypeStruct((N_NODES, H), jnp.float32),
    ]
    return pl.pallas_call(_node_body, out_shape=out_shape)(
        nf, agg2, w1n, w1a, b1, w2, b2, gamma, beta, w1s_nxt, w1d_nxt)


def _proj_body(nf_ref, w1s_ref, w1d_ref, ps_out, pd_out):
    x = nf_ref[...]
    ps_out[...] = jnp.dot(x, w1s_ref[...], preferred_element_type=jnp.float32)
    pd_out[...] = jnp.dot(x, w1d_ref[...], preferred_element_type=jnp.float32)


def _proj(nf, w1s, w1d):
    out_shape = [
        jax.ShapeDtypeStruct((N_NODES, H), jnp.float32),
        jax.ShapeDtypeStruct((N_NODES, H), jnp.float32),
    ]
    return pl.pallas_call(_proj_body, out_shape=out_shape)(nf, w1s, w1d)


def _make_gather():
    info = plsc.get_sparse_core_info()
    nc, ns = info.num_cores, info.num_subcores
    nw = nc * ns
    mesh = plsc.VectorSubcoreMesh(core_axis_name="c", subcore_axis_name="s")
    GB = 200
    E_PER = N_EDGES // nw
    NIT = E_PER // GB
    SUB = ((0, 128), (128, 72))
    CHUNKS = GB * H // 16

    @functools.partial(
        pl.kernel,
        mesh=mesh,
        out_type=jax.ShapeDtypeStruct((N_EDGES, H), jnp.float32),
        scratch_types=[
            pltpu.VMEM((E_PER,), jnp.int32),
            pltpu.VMEM((E_PER,), jnp.int32),
            pltpu.VMEM((GB, H), jnp.float32),
            pltpu.VMEM((GB, H), jnp.float32),
            pltpu.VMEM((GB, H), jnp.float32),
            pltpu.VMEM((GB, H), jnp.float32),
            pltpu.SemaphoreType.DMA,
            pltpu.SemaphoreType.DMA,
            pltpu.SemaphoreType.DMA,
            pltpu.SemaphoreType.DMA,
        ],
    )
    def gather(ps_hbm, pd_hbm, src_hbm, dst_hbm, g_hbm,
               isrc, idst, a0, d0, a1, d1, g0, g1, w0, w1):
        cid = lax.axis_index("c")
        sid = lax.axis_index("s")
        wid = sid * nc + cid
        base_e = wid * E_PER
        abufs = (a0, a1)
        dbufs = (d0, d1)
        gsems = (g0, g1)
        wsems = (w0, w1)

        pltpu.sync_copy(src_hbm.at[pl.ds(base_e, E_PER)], isrc)
        pltpu.sync_copy(dst_hbm.at[pl.ds(base_e, E_PER)], idst)

        def fire(it, s):
            off = it * GB
            for (o, n) in SUB:
                pltpu.async_copy(ps_hbm.at[isrc.at[pl.ds(off + o, n)]],
                                 abufs[s].at[pl.ds(o, n)], gsems[s])
                pltpu.async_copy(pd_hbm.at[idst.at[pl.ds(off + o, n)]],
                                 dbufs[s].at[pl.ds(o, n)], gsems[s])

        def drain(it, s):
            for (o, n) in SUB:
                pltpu.make_async_copy(ps_hbm.at[pl.ds(0, n)],
                                      abufs[s].at[pl.ds(o, n)],
                                      gsems[s]).wait()
                pltpu.make_async_copy(pd_hbm.at[pl.ds(0, n)],
                                      dbufs[s].at[pl.ds(o, n)],
                                      gsems[s]).wait()
            a, dd = abufs[s], dbufs[s]

            @plsc.parallel_loop(0, GB, 1, unroll=4)
            def _adds(r):
                for cc in range(H // 16):
                    c = cc * 16
                    a[r, pl.ds(c, 16)] = a[r, pl.ds(c, 16)] + dd[r, pl.ds(c, 16)]
            pltpu.async_copy(a, g_hbm.at[pl.ds(base_e + it * GB, GB)],
                             wsems[s])

        def wait_w(s):
            pltpu.make_async_copy(g_hbm.at[pl.ds(base_e, GB)],
                                  abufs[s], wsems[s]).wait()

        fire(0, 0)

        def body(i, _):
            @pl.when(i > 0)
            def _():
                wait_w(1)

            fire(2 * i + 1, 1)
            drain(2 * i, 0)

            @pl.when(i < NIT // 2 - 1)
            def _():
                wait_w(0)
                fire(2 * i + 2, 0)

            drain(2 * i + 1, 1)
            return 0

        lax.fori_loop(0, NIT // 2, body, 0)
        wait_w(0)
        wait_w(1)

    return gather


def _make_scatter():
    info = plsc.get_sparse_core_info()
    nc, ns = info.num_cores, info.num_subcores
    nw = nc * ns
    rps = 624
    tail = N_NODES - rps * ns
    mesh = plsc.VectorSubcoreMesh(core_axis_name="c", subcore_axis_name="s")

    blk_per_w = NB // nw
    n_extra = NB - blk_per_w * nw

    @functools.partial(
        pl.kernel,
        mesh=mesh,
        out_type=jax.ShapeDtypeStruct((2, N_NODES, D), jnp.float32),
        scratch_types=[
            pltpu.VMEM((EB,), jnp.int32),
            pltpu.VMEM((EB,), jnp.int32),
            pltpu.VMEM((EB, D), jnp.float32),
            pltpu.VMEM((EB, D), jnp.float32),
            pltpu.VMEM_SHARED((N_NODES, D), jnp.float32),
            pltpu.SemaphoreType.DMA,
            pltpu.SemaphoreType.DMA,
        ],
    )
    def scatter(e_hbm, dst_hbm, zeros_hbm, out_hbm,
                di0, di1, rb0, rb1, acc, r0sem, r1sem):
        cid = lax.axis_index("c")
        sid = lax.axis_index("s")
        wid = sid * nc + cid
        r0 = sid * rps
        pltpu.sync_copy(zeros_hbm.at[pl.ds(r0, rps)], acc.at[pl.ds(r0, rps)])

        @pl.when(sid == 0)
        def _():
            pltpu.sync_copy(zeros_hbm.at[pl.ds(rps * ns, tail)],
                            acc.at[pl.ds(rps * ns, tail)])

        plsc.subcore_barrier()

        t0 = wid * blk_per_w

        def fire(t, di, rb, rsem):
            pltpu.async_copy(dst_hbm.at[pl.ds(t * EB, EB)], di, rsem)
            pltpu.async_copy(e_hbm.at[pl.ds(t * EB, EB)], rb, rsem)

        def scat(di, rb, rsem):
            pltpu.make_async_copy(dst_hbm.at[pl.ds(0, EB)], di, rsem).wait()
            pltpu.make_async_copy(e_hbm.at[pl.ds(0, EB)], rb, rsem).wait()
            pltpu.sync_copy(rb, acc.at[di], add=True)

        fire(t0, di0, rb0, r0sem)

        def body(i, _):
            fire(t0 + 2 * i + 1, di1, rb1, r1sem)
            scat(di0, rb0, r0sem)

            @pl.when(i < blk_per_w // 2 - 1)
            def _():
                fire(t0 + 2 * i + 2, di0, rb0, r0sem)

            scat(di1, rb1, r1sem)
            return 0

        lax.fori_loop(0, blk_per_w // 2, body, 0)

        @pl.when(wid < n_extra)
        def _():
            fire(nw * blk_per_w + wid, di0, rb0, r0sem)
            scat(di0, rb0, r0sem)

        plsc.subcore_barrier()
        pltpu.sync_copy(acc.at[pl.ds(r0, rps)],
                        out_hbm.at[cid, pl.ds(r0, rps)])

        @pl.when(sid == 0)
        def _():
            pltpu.sync_copy(acc.at[pl.ds(rps * ns, tail)],
                            out_hbm.at[cid, pl.ds(rps * ns, tail)])

    return scatter


def kernel(efeat, nfeat, edge_index, params):
    src = edge_index[0].astype(jnp.int32)
    dst = edge_index[1].astype(jnp.int32)

    gather = _make_gather()
    scatter = _make_scatter()
    zeros = jnp.zeros((N_NODES, D), jnp.float32)

    def prep(p):
        e, n = p['edge'], p['node']
        return dict(
            w1e=e['w1'][:D], w1s=e['w1'][D:2 * D], w1d=e['w1'][2 * D:],
            eb1=e['b1'].reshape(1, H), ew2=e['w2'],
            eb2=e['b2'].reshape(1, D), eg=e['gamma'].reshape(1, D),
            ebt=e['beta'].reshape(1, D),
            w1n=n['w1'][:D], w1a=n['w1'][D:],
            nb1=n['b1'].reshape(1, H), nw2=n['w2'],
            nb2=n['b2'].reshape(1, D), ng=n['gamma'].reshape(1, D),
            nbt=n['beta'].reshape(1, D),
        )

    ps_list = [prep(p) for p in params]
    nlayers = len(ps_list)

    ps, pd = _proj(nfeat, ps_list[0]['w1s'], ps_list[0]['w1d'])
    for l, q in enumerate(ps_list):
        g = gather(ps, pd, src, dst)
        efeat = _edge_mlp(efeat, g, q['w1e'], q['eb1'], q['ew2'],
                          q['eb2'], q['eg'], q['ebt'])
        agg2 = scatter(efeat, dst, zeros)
        nxt = ps_list[(l + 1) % nlayers]
        nfeat, ps, pd = _node_mlp(nfeat, agg2, q['w1n'], q['w1a'], q['nb1'],
                                  q['nw2'], q['nb2'], q['ng'], q['nbt'],
                                  nxt['w1s'], nxt['w1d'])
    return (efeat, nfeat)


# speedup vs baseline: 1.5049x; 1.0006x over previous
import functools

import jax
import jax.numpy as jnp
from jax import lax
from jax.experimental import pallas as pl
from jax.experimental.pallas import tpu as pltpu
from jax.experimental.pallas import tpu_sc as plsc

N_NODES = 10000
N_EDGES = 320000
D = 128
H = 128

EB = 128
NB = N_EDGES // EB


def _edge_body(ef_ref, g_ref, w1_ref, b1_ref, w2_ref, b2_ref,
               gm_ref, bt_ref, out_ref):
    x = ef_ref[...]
    h = jnp.dot(x, w1_ref[...], preferred_element_type=jnp.float32)
    h = h + g_ref[...] + b1_ref[...]
    h = h * jax.nn.sigmoid(h)
    y = jnp.dot(h, w2_ref[...], preferred_element_type=jnp.float32) + b2_ref[...]
    mu = jnp.mean(y, axis=-1, keepdims=True)
    var = jnp.mean((y - mu) ** 2, axis=-1, keepdims=True)
    y = (y - mu) * lax.rsqrt(var + 1e-5)
    out_ref[...] = y * gm_ref[...] + bt_ref[...] + x


def _edge_mlp(ef, g, w1e, b1, w2, b2, gamma, beta, blk=8000):
    grid = (N_EDGES // blk,)
    row = lambda i: (i, 0)
    fix = lambda i: (0, 0)
    return pl.pallas_call(
        _edge_body,
        grid=grid,
        in_specs=[
            pl.BlockSpec((blk, D), row),
            pl.BlockSpec((blk, H), row),
            pl.BlockSpec((D, H), fix),
            pl.BlockSpec((1, H), fix),
            pl.BlockSpec((H, D), fix),
            pl.BlockSpec((1, D), fix),
            pl.BlockSpec((1, D), fix),
            pl.BlockSpec((1, D), fix),
        ],
        out_specs=pl.BlockSpec((blk, D), row),
        out_shape=jax.ShapeDtypeStruct((N_EDGES, D), jnp.float32),
        compiler_params=pltpu.CompilerParams(
            dimension_semantics=("arbitrary",)),
    )(ef, g, w1e, b1, w2, b2, gamma, beta)


def _node_body(nf_ref, agg_ref, w1n_ref, w1a_ref, b1_ref, w2_ref, b2_ref,
               gm_ref, bt_ref, w1s_ref, w1d_ref,
               nf_out, ps_out, pd_out):
    x = nf_ref[...]
    a = agg_ref[0] + agg_ref[1]
    h = (jnp.dot(x, w1n_ref[...], preferred_element_type=jnp.float32)
         + jnp.dot(a, w1a_ref[...], preferred_element_type=jnp.float32)
         + b1_ref[...])
    h = h * jax.nn.sigmoid(h)
    y = jnp.dot(h, w2_ref[...], preferred_element_type=jnp.float32) + b2_ref[...]
    mu = jnp.mean(y, axis=-1, keepdims=True)
    var = jnp.mean((y - mu) ** 2, axis=-1, keepdims=True)
    y = (y - mu) * lax.rsqrt(var + 1e-5)
    y = y * gm_ref[...] + bt_ref[...] + x
    nf_out[...] = y
    ps_out[...] = jnp.dot(y, w1s_ref[...], preferred_element_type=jnp.float32)
    pd_out[...] = jnp.dot(y, w1d_ref[...], preferred_element_type=jnp.float32)


def _node_mlp(nf, agg2, w1n, w1a, b1, w2, b2, gamma, beta, w1s_nxt, w1d_nxt):
    out_shape = [
        jax.ShapeDtypeStruct((N_NODES, D), jnp.float32),
        jax.ShapeDtypeStruct((N_NODES, H), jnp.float32),
        jax.ShapeDtypeStruct((N_NODES, H), jnp.float32),
    ]
    return pl.pallas_call(_node_body, out_shape=out_shape)(
        nf, agg2, w1n, w1a, b1, w2, b2, gamma, beta, w1s_nxt, w1d_nxt)


def _proj_body(nf_ref, w1s_ref, w1d_ref, ps_out, pd_out):
    x = nf_ref[...]
    ps_out[...] = jnp.dot(x, w1s_ref[...], preferred_element_type=jnp.float32)
    pd_out[...] = jnp.dot(x, w1d_ref[...], preferred_element_type=jnp.float32)


def _proj(nf, w1s, w1d):
    out_shape = [
        jax.ShapeDtypeStruct((N_NODES, H), jnp.float32),
        jax.ShapeDtypeStruct((N_NODES, H), jnp.float32),
    ]
    return pl.pallas_call(_proj_body, out_shape=out_shape)(nf, w1s, w1d)


def _make_gather():
    info = plsc.get_sparse_core_info()
    nc, ns = info.num_cores, info.num_subcores
    nw = nc * ns
    mesh = plsc.VectorSubcoreMesh(core_axis_name="c", subcore_axis_name="s")
    GB = 200
    E_PER = N_EDGES // nw
    NIT = E_PER // GB
    SUB = ((0, 128), (128, 72))
    CHUNKS = GB * H // 16

    @functools.partial(
        pl.kernel,
        mesh=mesh,
        out_type=jax.ShapeDtypeStruct((N_EDGES, H), jnp.float32),
        scratch_types=[
            pltpu.VMEM((E_PER,), jnp.int32),
            pltpu.VMEM((E_PER,), jnp.int32),
            pltpu.VMEM((GB, H), jnp.float32),
            pltpu.VMEM((GB, H), jnp.float32),
            pltpu.VMEM((GB, H), jnp.float32),
            pltpu.VMEM((GB, H), jnp.float32),
            pltpu.SemaphoreType.DMA,
            pltpu.SemaphoreType.DMA,
            pltpu.SemaphoreType.DMA,
            pltpu.SemaphoreType.DMA,
        ],
    )
    def gather(ps_hbm, pd_hbm, src_hbm, dst_hbm, g_hbm,
               isrc, idst, a0, d0, a1, d1, g0, g1, w0, w1):
        cid = lax.axis_index("c")
        sid = lax.axis_index("s")
        wid = sid * nc + cid
        base_e = wid * E_PER
        abufs = (a0, a1)
        dbufs = (d0, d1)
        gsems = (g0, g1)
        wsems = (w0, w1)

        pltpu.sync_copy(src_hbm.at[pl.ds(base_e, E_PER)], isrc)
        pltpu.sync_copy(dst_hbm.at[pl.ds(base_e, E_PER)], idst)

        def fire(it, s):
            off = it * GB
            for (o, n) in SUB:
                pltpu.async_copy(ps_hbm.at[isrc.at[pl.ds(off + o, n)]],
                                 abufs[s].at[pl.ds(o, n)], gsems[s])
                pltpu.async_copy(pd_hbm.at[idst.at[pl.ds(off + o, n)]],
                                 dbufs[s].at[pl.ds(o, n)], gsems[s])

        def drain(it, s):
            for (o, n) in SUB:
                pltpu.make_async_copy(ps_hbm.at[pl.ds(0, n)],
                                      abufs[s].at[pl.ds(o, n)],
                                      gsems[s]).wait()
                pltpu.make_async_copy(pd_hbm.at[pl.ds(0, n)],
                                      dbufs[s].at[pl.ds(o, n)],
                                      gsems[s]).wait()
            a, dd = abufs[s], dbufs[s]

            @plsc.parallel_loop(0, GB, 1, unroll=4)
            def _adds(r):
                for cc in range(H // 16):
                    c = cc * 16
                    a[r, pl.ds(c, 16)] = a[r, pl.ds(c, 16)] + dd[r, pl.ds(c, 16)]
            pltpu.async_copy(a, g_hbm.at[pl.ds(base_e + it * GB, GB)],
                             wsems[s])

        def wait_w(s):
            pltpu.make_async_copy(g_hbm.at[pl.ds(base_e, GB)],
                                  abufs[s], wsems[s]).wait()

        fire(0, 0)

        def body(i, _):
            @pl.when(i > 0)
            def _():
                wait_w(1)

            fire(2 * i + 1, 1)
            drain(2 * i, 0)

            @pl.when(i < NIT // 2 - 1)
            def _():
                wait_w(0)
                fire(2 * i + 2, 0)

            drain(2 * i + 1, 1)
            return 0

        lax.fori_loop(0, NIT // 2, body, 0)
        wait_w(0)
        wait_w(1)

    return gather


def _make_scatter():
    info = plsc.get_sparse_core_info()
    nc, ns = info.num_cores, info.num_subcores
    nw = nc * ns
    rps = 624
    tail = N_NODES - rps * ns
    mesh = plsc.VectorSubcoreMesh(core_axis_name="c", subcore_axis_name="s")

    blk_per_w = NB // nw
    n_extra = NB - blk_per_w * nw

    @functools.partial(
        pl.kernel,
        mesh=mesh,
        out_type=jax.ShapeDtypeStruct((2, N_NODES, D), jnp.float32),
        scratch_types=[
            pltpu.VMEM((EB,), jnp.int32),
            pltpu.VMEM((EB,), jnp.int32),
            pltpu.VMEM((EB,), jnp.int32),
            pltpu.VMEM((EB, D), jnp.float32),
            pltpu.VMEM((EB, D), jnp.float32),
            pltpu.VMEM((EB, D), jnp.float32),
            pltpu.VMEM_SHARED((N_NODES, D), jnp.float32),
            pltpu.SemaphoreType.DMA,
            pltpu.SemaphoreType.DMA,
            pltpu.SemaphoreType.DMA,
        ],
    )
    def scatter(e_hbm, dst_hbm, zeros_hbm, out_hbm,
                di0, di1, di2, rb0, rb1, rb2, acc, r0sem, r1sem, r2sem):
        cid = lax.axis_index("c")
        sid = lax.axis_index("s")
        wid = sid * nc + cid
        r0 = sid * rps
        pltpu.sync_copy(zeros_hbm.at[pl.ds(r0, rps)], acc.at[pl.ds(r0, rps)])

        @pl.when(sid == 0)
        def _():
            pltpu.sync_copy(zeros_hbm.at[pl.ds(rps * ns, tail)],
                            acc.at[pl.ds(rps * ns, tail)])

        plsc.subcore_barrier()

        t0 = wid * blk_per_w
        dis = (di0, di1, di2)
        rbs = (rb0, rb1, rb2)
        rsems = (r0sem, r1sem, r2sem)

        def fire(t, s):
            pltpu.async_copy(dst_hbm.at[pl.ds(t * EB, EB)], dis[s], rsems[s])
            pltpu.async_copy(e_hbm.at[pl.ds(t * EB, EB)], rbs[s], rsems[s])

        def scat(s):
            pltpu.make_async_copy(dst_hbm.at[pl.ds(0, EB)], dis[s],
                                  rsems[s]).wait()
            pltpu.make_async_copy(e_hbm.at[pl.ds(0, EB)], rbs[s],
                                  rsems[s]).wait()
            pltpu.sync_copy(rbs[s], acc.at[dis[s]], add=True)

        def body(i, _):
            for k in range(3):
                def _step(k=k):
                    def _scat():
                        scat((k + 1) % 3)

                    if k < 2:
                        pl.when(i > 0)(_scat)
                    else:
                        _scat()
                    fire(t0 + 3 * i + k, k)
                _step()
            return 0

        lax.fori_loop(0, blk_per_w // 3, body, 0)
        scat((blk_per_w - 2) % 3)
        scat((blk_per_w - 1) % 3)

        @pl.when(wid < n_extra)
        def _():
            fire(nw * blk_per_w + wid, 0)
            scat(0)

        plsc.subcore_barrier()
        pltpu.sync_copy(acc.at[pl.ds(r0, rps)],
                        out_hbm.at[cid, pl.ds(r0, rps)])

        @pl.when(sid == 0)
        def _():
            pltpu.sync_copy(acc.at[pl.ds(rps * ns, tail)],
                            out_hbm.at[cid, pl.ds(rps * ns, tail)])

    return scatter


def kernel(efeat, nfeat, edge_index, params):
    src = edge_index[0].astype(jnp.int32)
    dst = edge_index[1].astype(jnp.int32)

    gather = _make_gather()
    scatter = _make_scatter()
    zeros = jnp.zeros((N_NODES, D), jnp.float32)

    def prep(p):
        e, n = p['edge'], p['node']
        return dict(
            w1e=e['w1'][:D], w1s=e['w1'][D:2 * D], w1d=e['w1'][2 * D:],
            eb1=e['b1'].reshape(1, H), ew2=e['w2'],
            eb2=e['b2'].reshape(1, D), eg=e['gamma'].reshape(1, D),
            ebt=e['beta'].reshape(1, D),
            w1n=n['w1'][:D], w1a=n['w1'][D:],
            nb1=n['b1'].reshape(1, H), nw2=n['w2'],
            nb2=n['b2'].reshape(1, D), ng=n['gamma'].reshape(1, D),
            nbt=n['beta'].reshape(1, D),
        )

    ps_list = [prep(p) for p in params]
    nlayers = len(ps_list)

    ps, pd = _proj(nfeat, ps_list[0]['w1s'], ps_list[0]['w1d'])
    for l, q in enumerate(ps_list):
        g = gather(ps, pd, src, dst)
        efeat = _edge_mlp(efeat, g, q['w1e'], q['eb1'], q['ew2'],
                          q['eb2'], q['eg'], q['ebt'])
        agg2 = scatter(efeat, dst, zeros)
        nxt = ps_list[(l + 1) % nlayers]
        nfeat, ps, pd = _node_mlp(nfeat, agg2, q['w1n'], q['w1a'], q['nb1'],
                                  q['nw2'], q['nb2'], q['ng'], q['nbt'],
                                  nxt['w1s'], nxt['w1d'])
    return (efeat, nfeat)


# speedup vs baseline: 1.5054x; 1.0004x over previous
import functools

import jax
import jax.numpy as jnp
from jax import lax
from jax.experimental import pallas as pl
from jax.experimental.pallas import tpu as pltpu
from jax.experimental.pallas import tpu_sc as plsc

N_NODES = 10000
N_EDGES = 320000
D = 128
H = 128

EB = 128
NB = N_EDGES // EB


def _edge_body(ef_ref, g_ref, w1_ref, b1_ref, w2_ref, b2_ref,
               gm_ref, bt_ref, out_ref):
    x = ef_ref[...]
    h = jnp.dot(x, w1_ref[...], preferred_element_type=jnp.float32)
    h = h + g_ref[...] + b1_ref[...]
    h = h * jax.nn.sigmoid(h)
    y = jnp.dot(h, w2_ref[...], preferred_element_type=jnp.float32) + b2_ref[...]
    mu = jnp.mean(y, axis=-1, keepdims=True)
    var = jnp.mean((y - mu) ** 2, axis=-1, keepdims=True)
    y = (y - mu) * lax.rsqrt(var + 1e-5)
    out_ref[...] = y * gm_ref[...] + bt_ref[...] + x


def _edge_mlp(ef, g, w1e, b1, w2, b2, gamma, beta, blk=10000):
    grid = (N_EDGES // blk,)
    row = lambda i: (i, 0)
    fix = lambda i: (0, 0)
    return pl.pallas_call(
        _edge_body,
        grid=grid,
        in_specs=[
            pl.BlockSpec((blk, D), row),
            pl.BlockSpec((blk, H), row),
            pl.BlockSpec((D, H), fix),
            pl.BlockSpec((1, H), fix),
            pl.BlockSpec((H, D), fix),
            pl.BlockSpec((1, D), fix),
            pl.BlockSpec((1, D), fix),
            pl.BlockSpec((1, D), fix),
        ],
        out_specs=pl.BlockSpec((blk, D), row),
        out_shape=jax.ShapeDtypeStruct((N_EDGES, D), jnp.float32),
        compiler_params=pltpu.CompilerParams(
            dimension_semantics=("arbitrary",)),
    )(ef, g, w1e, b1, w2, b2, gamma, beta)


def _node_body(nf_ref, agg_ref, w1n_ref, w1a_ref, b1_ref, w2_ref, b2_ref,
               gm_ref, bt_ref, w1s_ref, w1d_ref,
               nf_out, ps_out, pd_out):
    x = nf_ref[...]
    a = agg_ref[0] + agg_ref[1]
    h = (jnp.dot(x, w1n_ref[...], preferred_element_type=jnp.float32)
         + jnp.dot(a, w1a_ref[...], preferred_element_type=jnp.float32)
         + b1_ref[...])
    h = h * jax.nn.sigmoid(h)
    y = jnp.dot(h, w2_ref[...], preferred_element_type=jnp.float32) + b2_ref[...]
    mu = jnp.mean(y, axis=-1, keepdims=True)
    var = jnp.mean((y - mu) ** 2, axis=-1, keepdims=True)
    y = (y - mu) * lax.rsqrt(var + 1e-5)
    y = y * gm_ref[...] + bt_ref[...] + x
    nf_out[...] = y
    ps_out[...] = jnp.dot(y, w1s_ref[...], preferred_element_type=jnp.float32)
    pd_out[...] = jnp.dot(y, w1d_ref[...], preferred_element_type=jnp.float32)


def _node_mlp(nf, agg2, w1n, w1a, b1, w2, b2, gamma, beta, w1s_nxt, w1d_nxt):
    out_shape = [
        jax.ShapeDtypeStruct((N_NODES, D), jnp.float32),
        jax.ShapeDtypeStruct((N_NODES, H), jnp.float32),
        jax.ShapeDtypeStruct((N_NODES, H), jnp.float32),
    ]
    return pl.pallas_call(_node_body, out_shape=out_shape)(
        nf, agg2, w1n, w1a, b1, w2, b2, gamma, beta, w1s_nxt, w1d_nxt)


def _proj_body(nf_ref, w1s_ref, w1d_ref, ps_out, pd_out):
    x = nf_ref[...]
    ps_out[...] = jnp.dot(x, w1s_ref[...], preferred_element_type=jnp.float32)
    pd_out[...] = jnp.dot(x, w1d_ref[...], preferred_element_type=jnp.float32)


def _proj(nf, w1s, w1d):
    out_shape = [
        jax.ShapeDtypeStruct((N_NODES, H), jnp.float32),
        jax.ShapeDtypeStruct((N_NODES, H), jnp.float32),
    ]
    return pl.pallas_call(_proj_body, out_shape=out_shape)(nf, w1s, w1d)


def _make_gather():
    info = plsc.get_sparse_core_info()
    nc, ns = info.num_cores, info.num_subcores
    nw = nc * ns
    mesh = plsc.VectorSubcoreMesh(core_axis_name="c", subcore_axis_name="s")
    GB = 200
    E_PER = N_EDGES // nw
    NIT = E_PER // GB
    SUB = ((0, 128), (128, 72))
    CHUNKS = GB * H // 16

    @functools.partial(
        pl.kernel,
        mesh=mesh,
        out_type=jax.ShapeDtypeStruct((N_EDGES, H), jnp.float32),
        scratch_types=[
            pltpu.VMEM((E_PER,), jnp.int32),
            pltpu.VMEM((E_PER,), jnp.int32),
            pltpu.VMEM((GB, H), jnp.float32),
            pltpu.VMEM((GB, H), jnp.float32),
            pltpu.VMEM((GB, H), jnp.float32),
            pltpu.VMEM((GB, H), jnp.float32),
            pltpu.SemaphoreType.DMA,
            pltpu.SemaphoreType.DMA,
            pltpu.SemaphoreType.DMA,
            pltpu.SemaphoreType.DMA,
        ],
    )
    def gather(ps_hbm, pd_hbm, src_hbm, dst_hbm, g_hbm,
               isrc, idst, a0, d0, a1, d1, g0, g1, w0, w1):
        cid = lax.axis_index("c")
        sid = lax.axis_index("s")
        wid = sid * nc + cid
        base_e = wid * E_PER
        abufs = (a0, a1)
        dbufs = (d0, d1)
        gsems = (g0, g1)
        wsems = (w0, w1)

        pltpu.sync_copy(src_hbm.at[pl.ds(base_e, E_PER)], isrc)
        pltpu.sync_copy(dst_hbm.at[pl.ds(base_e, E_PER)], idst)

        def fire(it, s):
            off = it * GB
            for (o, n) in SUB:
                pltpu.async_copy(ps_hbm.at[isrc.at[pl.ds(off + o, n)]],
                                 abufs[s].at[pl.ds(o, n)], gsems[s])
                pltpu.async_copy(pd_hbm.at[idst.at[pl.ds(off + o, n)]],
                                 dbufs[s].at[pl.ds(o, n)], gsems[s])

        def drain(it, s):
            for (o, n) in SUB:
                pltpu.make_async_copy(ps_hbm.at[pl.ds(0, n)],
                                      abufs[s].at[pl.ds(o, n)],
                                      gsems[s]).wait()
                pltpu.make_async_copy(pd_hbm.at[pl.ds(0, n)],
                                      dbufs[s].at[pl.ds(o, n)],
                                      gsems[s]).wait()
            a, dd = abufs[s], dbufs[s]

            @plsc.parallel_loop(0, GB, 1, unroll=4)
            def _adds(r):
                for cc in range(H // 16):
                    c = cc * 16
                    a[r, pl.ds(c, 16)] = a[r, pl.ds(c, 16)] + dd[r, pl.ds(c, 16)]
            pltpu.async_copy(a, g_hbm.at[pl.ds(base_e + it * GB, GB)],
                             wsems[s])

        def wait_w(s):
            pltpu.make_async_copy(g_hbm.at[pl.ds(base_e, GB)],
                                  abufs[s], wsems[s]).wait()

        fire(0, 0)

        def body(i, _):
            @pl.when(i > 0)
            def _():
                wait_w(1)

            fire(2 * i + 1, 1)
            drain(2 * i, 0)

            @pl.when(i < NIT // 2 - 1)
            def _():
                wait_w(0)
                fire(2 * i + 2, 0)

            drain(2 * i + 1, 1)
            return 0

        lax.fori_loop(0, NIT // 2, body, 0)
        wait_w(0)
        wait_w(1)

    return gather


def _make_scatter():
    info = plsc.get_sparse_core_info()
    nc, ns = info.num_cores, info.num_subcores
    nw = nc * ns
    rps = 624
    tail = N_NODES - rps * ns
    mesh = plsc.VectorSubcoreMesh(core_axis_name="c", subcore_axis_name="s")

    blk_per_w = NB // nw
    n_extra = NB - blk_per_w * nw

    @functools.partial(
        pl.kernel,
        mesh=mesh,
        out_type=jax.ShapeDtypeStruct((2, N_NODES, D), jnp.float32),
        scratch_types=[
            pltpu.VMEM((EB,), jnp.int32),
            pltpu.VMEM((EB,), jnp.int32),
            pltpu.VMEM((EB,), jnp.int32),
            pltpu.VMEM((EB, D), jnp.float32),
            pltpu.VMEM((EB, D), jnp.float32),
            pltpu.VMEM((EB, D), jnp.float32),
            pltpu.VMEM_SHARED((N_NODES, D), jnp.float32),
            pltpu.SemaphoreType.DMA,
            pltpu.SemaphoreType.DMA,
            pltpu.SemaphoreType.DMA,
        ],
    )
    def scatter(e_hbm, dst_hbm, zeros_hbm, out_hbm,
                di0, di1, di2, rb0, rb1, rb2, acc, r0sem, r1sem, r2sem):
        cid = lax.axis_index("c")
        sid = lax.axis_index("s")
        wid = sid * nc + cid
        r0 = sid * rps
        pltpu.sync_copy(zeros_hbm.at[pl.ds(r0, rps)], acc.at[pl.ds(r0, rps)])

        @pl.when(sid == 0)
        def _():
            pltpu.sync_copy(zeros_hbm.at[pl.ds(rps * ns, tail)],
                            acc.at[pl.ds(rps * ns, tail)])

        plsc.subcore_barrier()

        t0 = wid * blk_per_w
        dis = (di0, di1, di2)
        rbs = (rb0, rb1, rb2)
        rsems = (r0sem, r1sem, r2sem)

        def fire(t, s):
            pltpu.async_copy(dst_hbm.at[pl.ds(t * EB, EB)], dis[s], rsems[s])
            pltpu.async_copy(e_hbm.at[pl.ds(t * EB, EB)], rbs[s], rsems[s])

        def scat(s):
            pltpu.make_async_copy(dst_hbm.at[pl.ds(0, EB)], dis[s],
                                  rsems[s]).wait()
            pltpu.make_async_copy(e_hbm.at[pl.ds(0, EB)], rbs[s],
                                  rsems[s]).wait()
            pltpu.sync_copy(rbs[s], acc.at[dis[s]], add=True)

        def body(i, _):
            for k in range(3):
                def _step(k=k):
                    def _scat():
                        scat((k + 1) % 3)

                    if k < 2:
                        pl.when(i > 0)(_scat)
                    else:
                        _scat()
                    fire(t0 + 3 * i + k, k)
                _step()
            return 0

        lax.fori_loop(0, blk_per_w // 3, body, 0)
        scat((blk_per_w - 2) % 3)
        scat((blk_per_w - 1) % 3)

        @pl.when(wid < n_extra)
        def _():
            fire(nw * blk_per_w + wid, 0)
            scat(0)

        plsc.subcore_barrier()
        pltpu.sync_copy(acc.at[pl.ds(r0, rps)],
                        out_hbm.at[cid, pl.ds(r0, rps)])

        @pl.when(sid == 0)
        def _():
            pltpu.sync_copy(acc.at[pl.ds(rps * ns, tail)],
                            out_hbm.at[cid, pl.ds(rps * ns, tail)])

    return scatter


def kernel(efeat, nfeat, edge_index, params):
    src = edge_index[0].astype(jnp.int32)
    dst = edge_index[1].astype(jnp.int32)

    gather = _make_gather()
    scatter = _make_scatter()
    zeros = jnp.zeros((N_NODES, D), jnp.float32)

    def prep(p):
        e, n = p['edge'], p['node']
        return dict(
            w1e=e['w1'][:D], w1s=e['w1'][D:2 * D], w1d=e['w1'][2 * D:],
            eb1=e['b1'].reshape(1, H), ew2=e['w2'],
            eb2=e['b2'].reshape(1, D), eg=e['gamma'].reshape(1, D),
            ebt=e['beta'].reshape(1, D),
            w1n=n['w1'][:D], w1a=n['w1'][D:],
            nb1=n['b1'].reshape(1, H), nw2=n['w2'],
            nb2=n['b2'].reshape(1, D), ng=n['gamma'].reshape(1, D),
            nbt=n['beta'].reshape(1, D),
        )

    ps_list = [prep(p) for p in params]
    nlayers = len(ps_list)

    ps, pd = _proj(nfeat, ps_list[0]['w1s'], ps_list[0]['w1d'])
    for l, q in enumerate(ps_list):
        g = gather(ps, pd, src, dst)
        efeat = _edge_mlp(efeat, g, q['w1e'], q['eb1'], q['ew2'],
                          q['eb2'], q['eg'], q['ebt'])
        agg2 = scatter(efeat, dst, zeros)
        nxt = ps_list[(l + 1) % nlayers]
        nfeat, ps, pd = _node_mlp(nfeat, agg2, q['w1n'], q['w1a'], q['nb1'],
                                  q['nw2'], q['nb2'], q['ng'], q['nbt'],
                                  nxt['w1s'], nxt['w1d'])
    return (efeat, nfeat)
